# Initial kernel scaffold; baseline (speedup 1.0000x reference)
import jax
import jax.numpy as jnp
from jax.experimental import pallas as pl


def kernel(inputs, embeddings):
    raise NotImplementedError("write your pallas kernel here")


# speedup vs baseline: 6.1289x; 6.1289x over previous
import functools

import jax
import jax.numpy as jnp
from jax import lax
from jax.experimental import pallas as pl
from jax.experimental.pallas import tpu as pltpu
from jax.experimental.pallas import tpu_sc as plsc

D = 32
NC = 2
NS = 16
NW = NC * NS
SUB = 128
CHUNK_ROWS = 8
CHUNK = SUB * CHUNK_ROWS


@functools.lru_cache(maxsize=None)
def _make_lookup(n_flat):
    assert n_flat % (NW * CHUNK) == 0
    b_per_w = n_flat // NW
    n_chunks = b_per_w // CHUNK
    idx_rows_per_w = b_per_w // SUB
    mesh = plsc.VectorSubcoreMesh(core_axis_name="c", subcore_axis_name="s")

    @functools.partial(
        pl.kernel,
        out_type=jax.ShapeDtypeStruct((n_flat, D), jnp.float32),
        mesh=mesh,
        scratch_types=[
            pltpu.VMEM((CHUNK_ROWS, SUB), jnp.int32),
            pltpu.VMEM((CHUNK, D), jnp.float32),
            pltpu.SemaphoreType.DMA,
        ],
        compiler_params=pltpu.CompilerParams(use_tc_tiling_on_sc=False),
    )
    def lookup(idx_hbm, table_hbm, out_hbm, idx_v, rows_v, gsem):
        wid = lax.axis_index("s") * NC + lax.axis_index("c")
        row_base = wid * idx_rows_per_w
        flat_base = wid * b_per_w

        def chunk_body(c, carry):
            pltpu.sync_copy(
                idx_hbm.at[pl.ds(row_base + c * CHUNK_ROWS, CHUNK_ROWS)],
                idx_v,
            )
            copies = []
            for j in range(CHUNK_ROWS):
                copies.append(
                    pltpu.async_copy(
                        table_hbm.at[idx_v.at[j]],
                        rows_v.at[pl.ds(j * SUB, SUB)],
                        gsem,
                    )
                )
            for cp in copies:
                cp.wait()
            pltpu.sync_copy(
                rows_v,
                out_hbm.at[pl.ds(flat_base + c * CHUNK, CHUNK)],
            )
            return carry

        lax.fori_loop(0, n_chunks, chunk_body, 0)

    return lookup


def kernel(inputs, embeddings):
    b, h = inputs.shape
    n = b * h
    idx2d = inputs.reshape(n // SUB, SUB)
    out = _make_lookup(n)(idx2d, embeddings)
    return out.reshape(b, h, D)


# speedup vs baseline: 6.2795x; 1.0246x over previous
import functools

import jax
import jax.numpy as jnp
from jax import lax
from jax.experimental import pallas as pl
from jax.experimental.pallas import tpu as pltpu
from jax.experimental.pallas import tpu_sc as plsc

D = 32
NC = 2
NS = 16
NW = NC * NS
SUB = 128
CHUNK_ROWS = 8
CHUNK = SUB * CHUNK_ROWS
GROUP_ROWS = 2 * CHUNK_ROWS


@functools.lru_cache(maxsize=None)
def _make_lookup(n_flat):
    assert n_flat % (NW * 2 * CHUNK) == 0
    b_per_w = n_flat // NW
    n_groups = b_per_w // (2 * CHUNK)
    idx_rows_per_w = b_per_w // SUB
    mesh = plsc.VectorSubcoreMesh(core_axis_name="c", subcore_axis_name="s")

    @functools.partial(
        pl.kernel,
        out_type=jax.ShapeDtypeStruct((n_flat, D), jnp.float32),
        mesh=mesh,
        scratch_types=[
            pltpu.VMEM((GROUP_ROWS, SUB), jnp.int32),
            pltpu.VMEM((CHUNK, D), jnp.float32),
            pltpu.VMEM((CHUNK, D), jnp.float32),
            pltpu.SemaphoreType.DMA,
            pltpu.SemaphoreType.DMA,
        ],
        compiler_params=pltpu.CompilerParams(use_tc_tiling_on_sc=False),
    )
    def lookup(idx_hbm, table_hbm, out_hbm, idx_v, rows0, rows1, gsem, osem):
        wid = lax.axis_index("s") * NC + lax.axis_index("c")
        row_base = wid * idx_rows_per_w
        flat_base = wid * b_per_w

        def drain_one_store():
            pltpu.make_async_copy(
                rows0, out_hbm.at[pl.ds(flat_base, CHUNK)], osem
            ).wait()

        def half(g, b, rows_v, do_wait):
            if do_wait:
                drain_one_store()
            copies = [
                pltpu.async_copy(
                    table_hbm.at[idx_v.at[CHUNK_ROWS * b + j]],
                    rows_v.at[pl.ds(j * SUB, SUB)],
                    gsem,
                )
                for j in range(CHUNK_ROWS)
            ]
            for cp in copies:
                cp.wait()
            pltpu.async_copy(
                rows_v,
                out_hbm.at[pl.ds(flat_base + (2 * g + b) * CHUNK, CHUNK)],
                osem,
            )

        def group(g, do_wait):
            pltpu.sync_copy(
                idx_hbm.at[pl.ds(row_base + g * GROUP_ROWS, GROUP_ROWS)],
                idx_v,
            )
            half(g, 0, rows0, do_wait)
            half(g, 1, rows1, do_wait)

        group(0, False)

        def body(g, carry):
            group(g, True)
            return carry

        lax.fori_loop(1, n_groups, body, 0)
        drain_one_store()
        drain_one_store()

    return lookup


def kernel(inputs, embeddings):
    b, h = inputs.shape
    n = b * h
    idx2d = inputs.reshape(n // SUB, SUB)
    out = _make_lookup(n)(idx2d, embeddings)
    return out.reshape(b, h, D)
